# baseline (speedup 1.0000x reference)
import jax
import jax.numpy as jnp
from jax.experimental import pallas as pl


_ROWS = 131072
_COLS = 256
_BLOCK_ROWS = 4096


def _copy_body(x_ref, y_ref, ld_ref):
    y_ref[...] = x_ref[...]
    ld_ref[...] = jnp.zeros_like(ld_ref)


def kernel(inputs):
    n, c = inputs.shape
    block_rows = _BLOCK_ROWS if n % _BLOCK_ROWS == 0 else n
    grid = (n // block_rows,)
    outputs, ld = pl.pallas_call(
        _copy_body,
        grid=grid,
        in_specs=[pl.BlockSpec((block_rows, c), lambda i: (i, 0))],
        out_specs=[
            pl.BlockSpec((block_rows, c), lambda i: (i, 0)),
            pl.BlockSpec((block_rows,), lambda i: (i,)),
        ],
        out_shape=[
            jax.ShapeDtypeStruct((n, c), inputs.dtype),
            jax.ShapeDtypeStruct((n,), jnp.float32),
        ],
    )(inputs)
    return (outputs, ld)
